# baseline (speedup 1.0000x reference)
import functools

import jax
import jax.numpy as jnp
from jax import lax
from jax.experimental import pallas as pl
from jax.experimental.pallas import tpu as pltpu
from jax.experimental.pallas import tpu_sc as plsc

_B, _S, _D, _V = 4, 2048, 1024, 100000
_NC, _NS = 2, 16
_NW = _NC * _NS
_WS = _S // _NW
_C = 8
_UPP = _WS // _C
_NU = 2 * _UPP
_NB = 3


def _emb_body(idx_hbm, tok_hbm, wpe_hbm, out_hbm,
              idx_v, wpe_v, ta0, ta1, ta2, tb0, tb1, tb2,
              ga0, ga1, ga2, gb0, gb1, gb2,
              oa0, oa1, oa2, ob0, ob1, ob2, isem, wsem):
    toka = [ta0, ta1, ta2]
    tokb = [tb0, tb1, tb2]
    gsa = [ga0, ga1, ga2]
    gsb = [gb0, gb1, gb2]
    osa = [oa0, oa1, oa2]
    osb = [ob0, ob1, ob2]

    wid = lax.axis_index("s") * _NC + lax.axis_index("c")
    s_base = wid * _WS
    icps = [pltpu.async_copy(idx_hbm.at[pl.ds(b * _S + s_base, _WS)],
                             idx_v.at[pl.ds(b * _WS, _WS)], isem)
            for b in range(_B)]
    wcp = pltpu.async_copy(wpe_hbm.at[pl.ds(s_base, _WS)], wpe_v, wsem)
    for cp in icps:
        cp.wait()

    gca = [None] * _NB
    gcb = [None] * _NB
    oca = [None] * _NB
    ocb = [None] * _NB

    def issue(u):
        p = u % _NB
        bp, cc = u // _UPP, u % _UPP
        offa = (2 * bp) * _WS + cc * _C
        offb = (2 * bp + 1) * _WS + cc * _C
        gca[p] = pltpu.async_copy(
            tok_hbm.at[idx_v.at[pl.ds(offa, _C)]], toka[p], gsa[p])
        gcb[p] = pltpu.async_copy(
            tok_hbm.at[idx_v.at[pl.ds(offb, _C)]], tokb[p], gsb[p])

    def finish(u):
        p = u % _NB
        bp, cc = u // _UPP, u % _UPP
        gca[p].wait()
        gcb[p].wait()
        wrows = wpe_v.at[pl.ds(cc * _C, _C)]

        def _add_row(r, carry):
            def _add_half(h, carry2):
                base_j = h * (_D // 2)
                for j in range(_D // 32):
                    sl = pl.ds(base_j + j * 16, 16)
                    w = wrows[r, sl]
                    toka[p][r, sl] = toka[p][r, sl] + w
                    tokb[p][r, sl] = tokb[p][r, sl] + w
                return carry2

            lax.fori_loop(0, 2, _add_half, 0)
            return carry

        lax.fori_loop(0, _C, _add_row, 0)
        oca[p] = pltpu.async_copy(
            toka[p],
            out_hbm.at[pl.ds((2 * bp) * _S + s_base + cc * _C, _C)], osa[p])
        ocb[p] = pltpu.async_copy(
            tokb[p],
            out_hbm.at[pl.ds((2 * bp + 1) * _S + s_base + cc * _C, _C)],
            osb[p])

    issue(0)
    issue(1)
    wcp.wait()
    for u in range(2, _NU):
        p = u % _NB
        if oca[p] is not None:
            oca[p].wait()
            ocb[p].wait()
        issue(u)
        finish(u - 2)
    finish(_NU - 2)
    finish(_NU - 1)
    for p in range(_NB):
        oca[p].wait()
        ocb[p].wait()


_sc_embed = functools.partial(
    pl.kernel,
    out_type=jax.ShapeDtypeStruct((_B * _S, _D), jnp.float32),
    mesh=plsc.VectorSubcoreMesh(core_axis_name="c", subcore_axis_name="s"),
    scratch_types=(
        [pltpu.VMEM((_B * _WS,), jnp.int32),
         pltpu.VMEM((_WS, _D), jnp.float32)]
        + [pltpu.VMEM((_C, _D), jnp.float32)] * (2 * _NB)
        + [pltpu.SemaphoreType.DMA] * (4 * _NB + 2)
    ),
)(_emb_body)


def kernel(idx, tok_emb, wpe):
    flat = _sc_embed(idx.reshape(_B * _S), tok_emb, wpe)
    return flat.reshape(_B, _S, _D)


# speedup vs baseline: 1.7793x; 1.7793x over previous
import functools

import jax
import jax.numpy as jnp
from jax import lax
from jax.experimental import pallas as pl
from jax.experimental.pallas import tpu as pltpu
from jax.experimental.pallas import tpu_sc as plsc

_B, _S, _D, _V = 4, 2048, 1024, 100000
_NC, _NS = 2, 16
_NW = _NC * _NS
_WS = _S // _NW
_C = 16
_CPB = _WS // _C
_NCH = _B * _CPB
_NB = 3


def _emb_body(idx_hbm, tok_hbm, wpe_hbm, out_hbm,
              idx_v, wpe_v, tok0, tok1, tok2,
              g0, g1, g2, o0, o1, o2, isem, wsem):
    toks = [tok0, tok1, tok2]
    gsems = [g0, g1, g2]
    osems = [o0, o1, o2]

    wid = lax.axis_index("s") * _NC + lax.axis_index("c")
    s_base = wid * _WS
    icps = [pltpu.async_copy(idx_hbm.at[pl.ds(b * _S + s_base, _WS)],
                             idx_v.at[pl.ds(b * _WS, _WS)], isem)
            for b in range(_B)]
    wcp = pltpu.async_copy(wpe_hbm.at[pl.ds(s_base, _WS)], wpe_v, wsem)
    for cp in icps:
        cp.wait()

    gcp = [None] * _NB
    ocp = [None] * _NB

    def issue(t):
        p = t % _NB
        gcp[p] = pltpu.async_copy(
            tok_hbm.at[idx_v.at[pl.ds(t * _C, _C)]], toks[p], gsems[p])

    def finish(t):
        p = t % _NB
        b, cc = t // _CPB, t % _CPB
        gcp[p].wait()
        wrows = wpe_v.at[pl.ds(cc * _C, _C)]

        def _add_row(r, carry):
            for j in range(_D // 16):
                sl = pl.ds(j * 16, 16)
                toks[p][r, sl] = toks[p][r, sl] + wrows[r, sl]
            return carry

        lax.fori_loop(0, _C, _add_row, 0)
        ocp[p] = pltpu.async_copy(
            toks[p], out_hbm.at[pl.ds(b * _S + s_base + cc * _C, _C)],
            osems[p])

    issue(0)
    issue(1)
    wcp.wait()
    for t in range(2, _NCH):
        p = t % _NB
        if ocp[p] is not None:
            ocp[p].wait()
        issue(t)
        finish(t - 2)
    finish(_NCH - 2)
    finish(_NCH - 1)
    for p in range(_NB):
        ocp[p].wait()


_sc_embed = functools.partial(
    pl.kernel,
    out_type=jax.ShapeDtypeStruct((_B * _S, _D), jnp.float32),
    mesh=plsc.VectorSubcoreMesh(core_axis_name="c", subcore_axis_name="s"),
    scratch_types=(
        [pltpu.VMEM((_B * _WS,), jnp.int32),
         pltpu.VMEM((_WS, _D), jnp.float32)]
        + [pltpu.VMEM((_C, _D), jnp.float32)] * _NB
        + [pltpu.SemaphoreType.DMA] * (2 * _NB + 2)
    ),
)(_emb_body)


def kernel(idx, tok_emb, wpe):
    flat = _sc_embed(idx.reshape(_B * _S), tok_emb, wpe)
    return flat.reshape(_B, _S, _D)
